# baseline (speedup 1.0000x reference)
import functools

import jax
import jax.numpy as jnp
from jax import lax
from jax.experimental import pallas as pl
from jax.experimental.pallas import tpu as pltpu
from jax.experimental.pallas import tpu_sc as plsc

N = 10000
E = 320000
NF = 11
EF = 4
H = 128
G = 64

NC = 2
NS = 16
NW = NC * NS
EPW = E // NW
L = 16


NB = 5


def _edge_body(c_off, a_hbm, b_hbm, src_hbm, dst_hbm, c_hbm, out_hbm,
               a_v, b_v, src_v, dst_v, c_v, *banks):
    wid = lax.axis_index("s") * NC + lax.axis_index("c")
    base = wid * EPW
    pltpu.sync_copy(a_hbm, a_v)
    pltpu.sync_copy(b_hbm, b_v)
    pltpu.sync_copy(src_hbm.at[pl.ds(base, EPW)], src_v)
    pltpu.sync_copy(dst_hbm.at[pl.ds(base, EPW)], dst_v)
    pltpu.sync_copy(c_hbm.at[pl.ds(c_off + base, EPW)], c_v)

    zeros16 = jnp.zeros((L,), jnp.float32)

    @plsc.parallel_loop(0, N // L, 1, unroll=4)
    def _zero(i):
        sl = pl.ds(i * L, L)
        for k in range(NB):
            banks[k][sl] = zeros16

    def edge_step(i, carry):
        for k in range(NB):
            off = (i * NB + k) * L
            d16 = dst_v[pl.ds(off, L)]
            s16 = src_v[pl.ds(off, L)]
            cv = c_v[pl.ds(off, L)]
            av = plsc.load_gather(a_v, [d16])
            bv = plsc.load_gather(b_v, [s16])
            m = jnp.maximum(av + bv + cv, 0.0)
            plsc.addupdate_scatter(banks[k], [d16], m)
        return carry

    lax.fori_loop(0, EPW // (L * NB), edge_step, 0)

    @plsc.parallel_loop(0, N // L, 1, unroll=4)
    def _merge(i):
        sl = pl.ds(i * L, L)
        acc = banks[0][sl]
        for k in range(1, NB):
            acc = acc + banks[k][sl]
        banks[0][sl] = acc

    pltpu.sync_copy(banks[0], out_hbm.at[wid])


_edge_pass_cache = []


def _edge_pass(layer, a, b, src, dst, cflat):
    if not _edge_pass_cache:
        for l in range(3):
            _edge_pass_cache.append(pl.kernel(
                functools.partial(_edge_body, l * E),
                out_type=jax.ShapeDtypeStruct((NW, N), jnp.float32),
                mesh=plsc.VectorSubcoreMesh(core_axis_name="c",
                                            subcore_axis_name="s",
                                            num_cores=NC, num_subcores=NS),
                compiler_params=pltpu.CompilerParams(needs_layout_passes=False),
                scratch_types=[
                    pltpu.VMEM((N,), jnp.float32),
                    pltpu.VMEM((N,), jnp.float32),
                    pltpu.VMEM((EPW,), jnp.int32),
                    pltpu.VMEM((EPW,), jnp.int32),
                    pltpu.VMEM((EPW,), jnp.float32),
                ] + [pltpu.VMEM((N,), jnp.float32) for _ in range(NB)],
            ))
    return _edge_pass_cache[layer](a, b, src, dst, cflat)


def _row_dot(wT, m):
    return lax.dot_general(wT, m, (((1,), (1,)), ((), ())),
                           preferred_element_type=jnp.float32)[0]


def _pre_body(x_ref, eat_ref, wi_ref, wj_ref, wct_ref, bc_ref,
              a_out, b_out, c_out):
    x = x_ref[...]
    a_out[...] = _row_dot(wi_ref[...], x)
    b_out[...] = _row_dot(wj_ref[...], x)
    c = jnp.dot(wct_ref[...], eat_ref[...],
                preferred_element_type=jnp.float32) + bc_ref[...]
    for l in range(3):
        c_out[pl.ds(l * E, E)] = c[l]


def _bf16r(v):
    return v.astype(jnp.bfloat16).astype(jnp.float32)


def _agg_col(p):
    return lax.dot_general(p, jnp.ones((NW, 1), jnp.float32),
                           (((0,), (0,)), ((), ())),
                           preferred_element_type=jnp.float32,
                           precision=jax.lax.Precision.HIGHEST)


def _premm_body(x_ref, whx_ref, bh_ref, hpre_out):
    hpre_out[...] = (jnp.dot(x_ref[...], whx_ref[...],
                             preferred_element_type=jnp.float32)
                     + bh_ref[...])


def _layer_body(hpre_ref, p_ref, whl_ref, wi_ref, wj_ref,
                h_out, a_out, b_out):
    outer = _bf16r(_agg_col(p_ref[...])) * _bf16r(whl_ref[...])
    h = jnp.maximum(hpre_ref[...] + outer, 0.0)
    h_out[...] = h
    a_out[...] = _row_dot(wi_ref[...], h)
    b_out[...] = _row_dot(wj_ref[...], h)


def _final_body(hpre_ref, p_ref, whl_ref, batch_ref,
                wl_ref, bl_ref, wl2_ref, bl2_ref, out_ref):
    outer = _bf16r(_agg_col(p_ref[...])) * _bf16r(whl_ref[...])
    h = jnp.maximum(hpre_ref[...] + outer, 0.0)
    gid = lax.broadcasted_iota(jnp.int32, (G, N), 0)
    onehot = jnp.where(gid == batch_ref[...], 1.0, 0.0)
    g = jnp.dot(onehot, h, preferred_element_type=jnp.float32,
                precision=jax.lax.Precision.HIGHEST)
    g = jnp.maximum(
        jnp.dot(g, wl_ref[...], preferred_element_type=jnp.float32)
        + bl_ref[...], 0.0)
    out_ref[...] = (jnp.dot(g, wl2_ref[...], preferred_element_type=jnp.float32)
                    + bl2_ref[...])


def _f32(shape):
    return jax.ShapeDtypeStruct(shape, jnp.float32)


def kernel(x, edge_index, edge_attr, batch,
           We1, be1, Wh1, bh1, We2, be2, Wh2, bh2, We3, be3, Wh3, bh3,
           Wl, bl, Wl2, bl2):
    src = edge_index[0]
    dst = edge_index[1]

    wct = jnp.concatenate([We1[2 * NF:2 * NF + EF],
                           We2[2 * H:2 * H + EF],
                           We3[2 * H:2 * H + EF]], axis=1).T
    bc = jnp.concatenate([be1, be2, be3]).reshape(3, 1)

    a1, b1, cf = pl.pallas_call(
        _pre_body,
        out_shape=[_f32((N,)), _f32((N,)), _f32((3 * E,))],
    )(x, edge_attr.T, We1[0:NF].T, We1[NF:2 * NF].T, wct, bc)

    _premm = lambda xx, whx, bh: pl.pallas_call(
        _premm_body, out_shape=_f32((N, H)))(xx, whx, bh)

    p1 = _edge_pass(0, a1, b1, src, dst, cf)
    hpre1 = _premm(x, Wh1[:NF], bh1.reshape(1, H))

    h1, a2, b2 = pl.pallas_call(
        _layer_body,
        out_shape=[_f32((N, H)), _f32((N,)), _f32((N,))],
    )(hpre1, p1, Wh1[NF:NF + 1], We2[0:H].T, We2[H:2 * H].T)

    p2 = _edge_pass(1, a2, b2, src, dst, cf)
    hpre2 = _premm(h1, Wh2[:H], bh2.reshape(1, H))

    h2, a3, b3 = pl.pallas_call(
        _layer_body,
        out_shape=[_f32((N, H)), _f32((N,)), _f32((N,))],
    )(hpre2, p2, Wh2[H:H + 1], We3[0:H].T, We3[H:2 * H].T)

    p3 = _edge_pass(2, a3, b3, src, dst, cf)
    hpre3 = _premm(h2, Wh3[:H], bh3.reshape(1, H))

    out = pl.pallas_call(
        _final_body,
        out_shape=_f32((G, 1)),
    )(hpre3, p3, Wh3[H:H + 1],
      batch.reshape(1, N), Wl, bl.reshape(1, G), Wl2, bl2.reshape(1, 1))

    return out


# speedup vs baseline: 1.1025x; 1.1025x over previous
import functools

import jax
import jax.numpy as jnp
from jax import lax
from jax.experimental import pallas as pl
from jax.experimental.pallas import tpu as pltpu
from jax.experimental.pallas import tpu_sc as plsc

N = 10000
E = 320000
NF = 11
EF = 4
H = 128
G = 64

NC = 2
NS = 16
NW = NC * NS
EPW = E // NW
L = 16


NB = 5


def _edge_body(c_off, a_hbm, b_hbm, src_hbm, dst_hbm, c_hbm, out_hbm,
               a_v, b_v, src_v, dst_v, c_v, *banks):
    wid = lax.axis_index("s") * NC + lax.axis_index("c")
    base = wid * EPW
    pltpu.sync_copy(a_hbm, a_v)
    pltpu.sync_copy(b_hbm, b_v)
    pltpu.sync_copy(src_hbm.at[pl.ds(base, EPW)], src_v)
    pltpu.sync_copy(dst_hbm.at[pl.ds(base, EPW)], dst_v)
    pltpu.sync_copy(c_hbm.at[pl.ds(c_off + base, EPW)], c_v)

    zeros16 = jnp.zeros((L,), jnp.float32)

    @plsc.parallel_loop(0, N // L, 1, unroll=4)
    def _zero(i):
        sl = pl.ds(i * L, L)
        for k in range(NB):
            banks[k][sl] = zeros16

    @plsc.parallel_loop(0, EPW // (L * NB), 1, unroll=2)
    def _edges(i):
        for k in range(NB):
            off = (i * NB + k) * L
            d16 = dst_v[pl.ds(off, L)]
            s16 = src_v[pl.ds(off, L)]
            cv = c_v[pl.ds(off, L)]
            av = plsc.load_gather(a_v, [d16])
            bv = plsc.load_gather(b_v, [s16])
            m = jnp.maximum(av + bv + cv, 0.0)
            plsc.addupdate_scatter(banks[k], [d16], m)

    @plsc.parallel_loop(0, N // L, 1, unroll=4)
    def _merge(i):
        sl = pl.ds(i * L, L)
        acc = banks[0][sl]
        for k in range(1, NB):
            acc = acc + banks[k][sl]
        banks[0][sl] = acc

    pltpu.sync_copy(banks[0], out_hbm.at[wid])


_edge_pass_cache = []


def _edge_pass(layer, a, b, src, dst, cflat):
    if not _edge_pass_cache:
        for l in range(3):
            _edge_pass_cache.append(pl.kernel(
                functools.partial(_edge_body, l * E),
                out_type=jax.ShapeDtypeStruct((NW, N), jnp.float32),
                mesh=plsc.VectorSubcoreMesh(core_axis_name="c",
                                            subcore_axis_name="s",
                                            num_cores=NC, num_subcores=NS),
                compiler_params=pltpu.CompilerParams(needs_layout_passes=False),
                scratch_types=[
                    pltpu.VMEM((N,), jnp.float32),
                    pltpu.VMEM((N,), jnp.float32),
                    pltpu.VMEM((EPW,), jnp.int32),
                    pltpu.VMEM((EPW,), jnp.int32),
                    pltpu.VMEM((EPW,), jnp.float32),
                ] + [pltpu.VMEM((N,), jnp.float32) for _ in range(NB)],
            ))
    return _edge_pass_cache[layer](a, b, src, dst, cflat)


def _row_dot(wT, m):
    return lax.dot_general(wT, m, (((1,), (1,)), ((), ())),
                           preferred_element_type=jnp.float32)[0]


def _pre_body(x_ref, eat_ref, wi_ref, wj_ref, wct_ref, bc_ref,
              a_out, b_out, c_out):
    x = x_ref[...]
    a_out[...] = _row_dot(wi_ref[...], x)
    b_out[...] = _row_dot(wj_ref[...], x)
    c = jnp.dot(wct_ref[...], eat_ref[...],
                preferred_element_type=jnp.float32) + bc_ref[...]
    for l in range(3):
        c_out[pl.ds(l * E, E)] = c[l]


def _bf16r(v):
    return v.astype(jnp.bfloat16).astype(jnp.float32)


def _agg_col(p):
    return lax.dot_general(p, jnp.ones((NW, 1), jnp.float32),
                           (((0,), (0,)), ((), ())),
                           preferred_element_type=jnp.float32,
                           precision=jax.lax.Precision.HIGHEST)


def _premm_body(x_ref, whx_ref, bh_ref, hpre_out):
    hpre_out[...] = (jnp.dot(x_ref[...], whx_ref[...],
                             preferred_element_type=jnp.float32)
                     + bh_ref[...])


def _layer_body(hpre_ref, p_ref, whl_ref, wi_ref, wj_ref,
                h_out, a_out, b_out):
    outer = _bf16r(_agg_col(p_ref[...])) * _bf16r(whl_ref[...])
    h = jnp.maximum(hpre_ref[...] + outer, 0.0)
    h_out[...] = h
    a_out[...] = _row_dot(wi_ref[...], h)
    b_out[...] = _row_dot(wj_ref[...], h)


def _final_body(hpre_ref, p_ref, whl_ref, batch_ref,
                wl_ref, bl_ref, wl2_ref, bl2_ref, out_ref):
    outer = _bf16r(_agg_col(p_ref[...])) * _bf16r(whl_ref[...])
    h = jnp.maximum(hpre_ref[...] + outer, 0.0)
    gid = lax.broadcasted_iota(jnp.int32, (G, N), 0)
    onehot = jnp.where(gid == batch_ref[...], 1.0, 0.0)
    g = jnp.dot(onehot, h, preferred_element_type=jnp.float32,
                precision=jax.lax.Precision.HIGHEST)
    g = jnp.maximum(
        jnp.dot(g, wl_ref[...], preferred_element_type=jnp.float32)
        + bl_ref[...], 0.0)
    out_ref[...] = (jnp.dot(g, wl2_ref[...], preferred_element_type=jnp.float32)
                    + bl2_ref[...])


def _f32(shape):
    return jax.ShapeDtypeStruct(shape, jnp.float32)


def kernel(x, edge_index, edge_attr, batch,
           We1, be1, Wh1, bh1, We2, be2, Wh2, bh2, We3, be3, Wh3, bh3,
           Wl, bl, Wl2, bl2):
    src = edge_index[0]
    dst = edge_index[1]

    wct = jnp.concatenate([We1[2 * NF:2 * NF + EF],
                           We2[2 * H:2 * H + EF],
                           We3[2 * H:2 * H + EF]], axis=1).T
    bc = jnp.concatenate([be1, be2, be3]).reshape(3, 1)

    a1, b1, cf = pl.pallas_call(
        _pre_body,
        out_shape=[_f32((N,)), _f32((N,)), _f32((3 * E,))],
    )(x, edge_attr.T, We1[0:NF].T, We1[NF:2 * NF].T, wct, bc)

    _premm = lambda xx, whx, bh: pl.pallas_call(
        _premm_body, out_shape=_f32((N, H)))(xx, whx, bh)

    p1 = _edge_pass(0, a1, b1, src, dst, cf)
    hpre1 = _premm(x, Wh1[:NF], bh1.reshape(1, H))

    h1, a2, b2 = pl.pallas_call(
        _layer_body,
        out_shape=[_f32((N, H)), _f32((N,)), _f32((N,))],
    )(hpre1, p1, Wh1[NF:NF + 1], We2[0:H].T, We2[H:2 * H].T)

    p2 = _edge_pass(1, a2, b2, src, dst, cf)
    hpre2 = _premm(h1, Wh2[:H], bh2.reshape(1, H))

    h2, a3, b3 = pl.pallas_call(
        _layer_body,
        out_shape=[_f32((N, H)), _f32((N,)), _f32((N,))],
    )(hpre2, p2, Wh2[H:H + 1], We3[0:H].T, We3[H:2 * H].T)

    p3 = _edge_pass(2, a3, b3, src, dst, cf)
    hpre3 = _premm(h2, Wh3[:H], bh3.reshape(1, H))

    out = pl.pallas_call(
        _final_body,
        out_shape=_f32((G, 1)),
    )(hpre3, p3, Wh3[H:H + 1],
      batch.reshape(1, N), Wl, bl.reshape(1, G), Wl2, bl2.reshape(1, 1))

    return out


# speedup vs baseline: 1.1046x; 1.0019x over previous
import functools

import jax
import jax.numpy as jnp
from jax import lax
from jax.experimental import pallas as pl
from jax.experimental.pallas import tpu as pltpu
from jax.experimental.pallas import tpu_sc as plsc

N = 10000
E = 320000
NF = 11
EF = 4
H = 128
G = 64

NC = 2
NS = 16
NW = NC * NS
EPW = E // NW
L = 16


NB = 5


def _edge_body(c_off, a_hbm, b_hbm, src_hbm, dst_hbm, c_hbm, out_hbm,
               a_v, b_v, src_v, dst_v, c_v, *banks):
    wid = lax.axis_index("s") * NC + lax.axis_index("c")
    base = wid * EPW
    pltpu.sync_copy(a_hbm, a_v)
    pltpu.sync_copy(b_hbm, b_v)
    pltpu.sync_copy(src_hbm.at[pl.ds(base, EPW)], src_v)
    pltpu.sync_copy(dst_hbm.at[pl.ds(base, EPW)], dst_v)
    pltpu.sync_copy(c_hbm.at[pl.ds(c_off + base, EPW)], c_v)

    zeros16 = jnp.zeros((L,), jnp.float32)

    @plsc.parallel_loop(0, N // L, 1, unroll=4)
    def _zero(i):
        sl = pl.ds(i * L, L)
        for k in range(NB):
            banks[k][sl] = zeros16

    @plsc.parallel_loop(0, EPW // (L * NB), 1, unroll=4)
    def _edges(i):
        for k in range(NB):
            off = (i * NB + k) * L
            d16 = dst_v[pl.ds(off, L)]
            s16 = src_v[pl.ds(off, L)]
            cv = c_v[pl.ds(off, L)]
            av = plsc.load_gather(a_v, [d16])
            bv = plsc.load_gather(b_v, [s16])
            m = jnp.maximum(av + bv + cv, 0.0)
            plsc.addupdate_scatter(banks[k], [d16], m)

    @plsc.parallel_loop(0, N // L, 1, unroll=4)
    def _merge(i):
        sl = pl.ds(i * L, L)
        acc = banks[0][sl]
        for k in range(1, NB):
            acc = acc + banks[k][sl]
        banks[0][sl] = acc

    pltpu.sync_copy(banks[0], out_hbm.at[wid])


_edge_pass_cache = []


def _edge_pass(layer, a, b, src, dst, cflat):
    if not _edge_pass_cache:
        for l in range(3):
            _edge_pass_cache.append(pl.kernel(
                functools.partial(_edge_body, l * E),
                out_type=jax.ShapeDtypeStruct((NW, N), jnp.float32),
                mesh=plsc.VectorSubcoreMesh(core_axis_name="c",
                                            subcore_axis_name="s",
                                            num_cores=NC, num_subcores=NS),
                compiler_params=pltpu.CompilerParams(needs_layout_passes=False),
                scratch_types=[
                    pltpu.VMEM((N,), jnp.float32),
                    pltpu.VMEM((N,), jnp.float32),
                    pltpu.VMEM((EPW,), jnp.int32),
                    pltpu.VMEM((EPW,), jnp.int32),
                    pltpu.VMEM((EPW,), jnp.float32),
                ] + [pltpu.VMEM((N,), jnp.float32) for _ in range(NB)],
            ))
    return _edge_pass_cache[layer](a, b, src, dst, cflat)


def _row_dot(wT, m):
    return lax.dot_general(wT, m, (((1,), (1,)), ((), ())),
                           preferred_element_type=jnp.float32)[0]


def _pre_body(x_ref, eat_ref, wi_ref, wj_ref, wct_ref, bc_ref,
              a_out, b_out, c_out):
    x = x_ref[...]
    a_out[...] = _row_dot(wi_ref[...], x)
    b_out[...] = _row_dot(wj_ref[...], x)
    c = jnp.dot(wct_ref[...], eat_ref[...],
                preferred_element_type=jnp.float32) + bc_ref[...]
    for l in range(3):
        c_out[pl.ds(l * E, E)] = c[l]


def _bf16r(v):
    return v.astype(jnp.bfloat16).astype(jnp.float32)


def _agg_col(p):
    return lax.dot_general(p, jnp.ones((NW, 1), jnp.float32),
                           (((0,), (0,)), ((), ())),
                           preferred_element_type=jnp.float32,
                           precision=jax.lax.Precision.HIGHEST)


def _premm_body(x_ref, whx_ref, bh_ref, hpre_out):
    hpre_out[...] = (jnp.dot(x_ref[...], whx_ref[...],
                             preferred_element_type=jnp.float32)
                     + bh_ref[...])


def _layer_body(hpre_ref, p_ref, whl_ref, wi_ref, wj_ref,
                h_out, a_out, b_out):
    outer = _bf16r(_agg_col(p_ref[...])) * _bf16r(whl_ref[...])
    h = jnp.maximum(hpre_ref[...] + outer, 0.0)
    h_out[...] = h
    a_out[...] = _row_dot(wi_ref[...], h)
    b_out[...] = _row_dot(wj_ref[...], h)


def _final_body(hpre_ref, p_ref, whl_ref, batch_ref,
                wl_ref, bl_ref, wl2_ref, bl2_ref, out_ref):
    outer = _bf16r(_agg_col(p_ref[...])) * _bf16r(whl_ref[...])
    h = jnp.maximum(hpre_ref[...] + outer, 0.0)
    gid = lax.broadcasted_iota(jnp.int32, (G, N), 0)
    onehot = jnp.where(gid == batch_ref[...], 1.0, 0.0)
    g = jnp.dot(onehot, h, preferred_element_type=jnp.float32,
                precision=jax.lax.Precision.HIGHEST)
    g = jnp.maximum(
        jnp.dot(g, wl_ref[...], preferred_element_type=jnp.float32)
        + bl_ref[...], 0.0)
    out_ref[...] = (jnp.dot(g, wl2_ref[...], preferred_element_type=jnp.float32)
                    + bl2_ref[...])


def _f32(shape):
    return jax.ShapeDtypeStruct(shape, jnp.float32)


def kernel(x, edge_index, edge_attr, batch,
           We1, be1, Wh1, bh1, We2, be2, Wh2, bh2, We3, be3, Wh3, bh3,
           Wl, bl, Wl2, bl2):
    src = edge_index[0]
    dst = edge_index[1]

    wct = jnp.concatenate([We1[2 * NF:2 * NF + EF],
                           We2[2 * H:2 * H + EF],
                           We3[2 * H:2 * H + EF]], axis=1).T
    bc = jnp.concatenate([be1, be2, be3]).reshape(3, 1)

    a1, b1, cf = pl.pallas_call(
        _pre_body,
        out_shape=[_f32((N,)), _f32((N,)), _f32((3 * E,))],
    )(x, edge_attr.T, We1[0:NF].T, We1[NF:2 * NF].T, wct, bc)

    _premm = lambda xx, whx, bh: pl.pallas_call(
        _premm_body, out_shape=_f32((N, H)))(xx, whx, bh)

    p1 = _edge_pass(0, a1, b1, src, dst, cf)
    hpre1 = _premm(x, Wh1[:NF], bh1.reshape(1, H))

    h1, a2, b2 = pl.pallas_call(
        _layer_body,
        out_shape=[_f32((N, H)), _f32((N,)), _f32((N,))],
    )(hpre1, p1, Wh1[NF:NF + 1], We2[0:H].T, We2[H:2 * H].T)

    p2 = _edge_pass(1, a2, b2, src, dst, cf)
    hpre2 = _premm(h1, Wh2[:H], bh2.reshape(1, H))

    h2, a3, b3 = pl.pallas_call(
        _layer_body,
        out_shape=[_f32((N, H)), _f32((N,)), _f32((N,))],
    )(hpre2, p2, Wh2[H:H + 1], We3[0:H].T, We3[H:2 * H].T)

    p3 = _edge_pass(2, a3, b3, src, dst, cf)
    hpre3 = _premm(h2, Wh3[:H], bh3.reshape(1, H))

    out = pl.pallas_call(
        _final_body,
        out_shape=_f32((G, 1)),
    )(hpre3, p3, Wh3[H:H + 1],
      batch.reshape(1, N), Wl, bl.reshape(1, G), Wl2, bl2.reshape(1, 1))

    return out
